# baseline (speedup 1.0000x reference)
import functools

import jax
import jax.numpy as jnp
from jax import lax
from jax.experimental import pallas as pl
from jax.experimental.pallas import tpu as pltpu
from jax.experimental.pallas import tpu_sc as plsc

D = 768
HALF = D // 2
B = 1024
S = 50
SG = S + 1
SRUN = 64
TAIL = 8
NW = 32
ROWS_PER_W = B // NW
TC_BLK = 256


def _tc_body(t_ref, c_ref, w_ref, temb_ref, cond_ref):
    t = t_ref[:]
    k = lax.broadcasted_iota(jnp.int32, (1, HALF), 1).astype(jnp.float32)
    inv_freq = jnp.exp(k * (-2.0 * jnp.log(100.0) / D))
    arg = t * inv_freq
    temb_ref[:, :HALF] = jnp.sin(arg)
    temb_ref[:, HALF:] = jnp.cos(arg)
    cond_ref[:] = lax.dot_general(
        c_ref[:], w_ref[:], (((1,), (1,)), ((), ())),
        preferred_element_type=jnp.float32)


def _tc_call(t2, cond_emb, w):
    return pl.pallas_call(
        _tc_body,
        grid=(B // TC_BLK,),
        in_specs=[
            pl.BlockSpec((TC_BLK, 1), lambda i: (i, 0)),
            pl.BlockSpec((TC_BLK, D), lambda i: (i, 0)),
            pl.BlockSpec((D, D), lambda i: (0, 0)),
        ],
        out_specs=[
            pl.BlockSpec((TC_BLK, D), lambda i: (i, 0)),
            pl.BlockSpec((TC_BLK, D), lambda i: (i, 0)),
        ],
        out_shape=[
            jax.ShapeDtypeStruct((B, D), jnp.float32),
            jax.ShapeDtypeStruct((B, D), jnp.float32),
        ],
    )(t2, cond_emb, w)


_mesh = plsc.VectorSubcoreMesh(core_axis_name="c", subcore_axis_name="s")

NCHUNK = 2
BC = B // NCHUNK
ROWS_PER_W_C = BC // NW


@functools.partial(
    pl.kernel,
    mesh=_mesh,
    out_type=jax.ShapeDtypeStruct((BC, SG, D), jnp.float32),
    scratch_types=[
        pltpu.VMEM((ROWS_PER_W_C * SRUN,), jnp.int32),
        pltpu.VMEM((SG, D), jnp.float32),
        pltpu.VMEM((SG, D), jnp.float32),
        pltpu.VMEM((TAIL, D), jnp.float32),
        pltpu.VMEM((TAIL, D), jnp.float32),
        pltpu.SemaphoreType.DMA,
        pltpu.SemaphoreType.DMA,
        pltpu.SemaphoreType.DMA,
        pltpu.SemaphoreType.DMA,
        pltpu.SemaphoreType.DMA,
        pltpu.SemaphoreType.DMA,
    ],
)
def _sc_gather(xg_hbm, temb1_hbm, table_hbm, out_hbm,
               idxs_v, bw0, bw1, bt0, bt1, g0, g1, t0, t1, w0, w1):
    wid = lax.axis_index("s") * 2 + lax.axis_index("c")
    base = wid * ROWS_PER_W_C
    bufw = (bw0, bw1)
    buft = (bt0, bt1)
    gsems = (g0, g1)
    tsems = (t0, t1)
    wsems = (w0, w1)

    pltpu.sync_copy(xg_hbm.at[pl.ds(base * SRUN, ROWS_PER_W_C * SRUN)], idxs_v)

    def issue_gathers(i, p):
        pltpu.async_copy(table_hbm.at[idxs_v.at[pl.ds(i * SRUN, SG)]],
                         bufw[p], gsems[p])
        pltpu.async_copy(table_hbm.at[idxs_v.at[pl.ds(i * SRUN + 56, TAIL)]],
                         buft[p], tsems[p])

    def wait_write(p):
        pltpu.make_async_copy(bufw[p], out_hbm.at[0], wsems[p]).wait()

    def drain_and_write(i, p):
        pltpu.make_async_copy(table_hbm.at[idxs_v.at[pl.ds(0, SG)]],
                              bufw[p], gsems[p]).wait()
        pltpu.make_async_copy(table_hbm.at[idxs_v.at[pl.ds(0, TAIL)]],
                              buft[p], tsems[p]).wait()
        for r in range(3):
            for c in range(D // 16):
                bufw[p][48 + r, pl.ds(c * 16, 16)] = \
                    buft[p][r, pl.ds(c * 16, 16)]
        pltpu.sync_copy(temb1_hbm.at[pl.ds((base + i) * D, D)],
                        bufw[p].at[0])
        pltpu.async_copy(bufw[p], out_hbm.at[base + i], wsems[p])

    issue_gathers(0, 0)
    issue_gathers(1, 1)

    @pl.loop(0, ROWS_PER_W_C - 2, step=2)
    def _(g):
        drain_and_write(g, 0)
        drain_and_write(g + 1, 1)
        wait_write(0)
        issue_gathers(g + 2, 0)
        wait_write(1)
        issue_gathers(g + 3, 1)

    drain_and_write(ROWS_PER_W_C - 2, 0)
    drain_and_write(ROWS_PER_W_C - 1, 1)
    wait_write(0)
    wait_write(1)


def kernel(x, t, condition_emb, emb_table, cond_W):
    x2 = x.astype(jnp.int32)
    d5 = jnp.tile(x2[:, :1], (1, 5))
    xg = jnp.concatenate([x2[:, :1], x2, d5, x2[:, 47:50], d5], axis=1)
    xg1 = xg.reshape(-1)
    temb, cond = _tc_call(t.reshape(B, 1), condition_emb, cond_W)
    temb1 = temb.reshape(-1)
    xgc = xg1.reshape(NCHUNK, BC * SRUN)
    tec = temb1.reshape(NCHUNK, BC * D)
    dit = jnp.zeros((B, SG, D), jnp.float32)
    for k in range(NCHUNK):
        dit = lax.dynamic_update_slice(
            dit, _sc_gather(xgc[k], tec[k], emb_table), (k * BC, 0, 0))
    return dit, cond


# speedup vs baseline: 1.4922x; 1.4922x over previous
import functools

import jax
import jax.numpy as jnp
from jax import lax
from jax.experimental import pallas as pl
from jax.experimental.pallas import tpu as pltpu
from jax.experimental.pallas import tpu_sc as plsc

D = 768
HALF = D // 2
B = 1024
S = 50
SG = S + 1
SRUN = 64
TAIL = 8
NW = 32
ROWS_PER_W = B // NW
TC_BLK = 256


def _tc_body(t_ref, c_ref, w_ref, temb_ref, cond_ref):
    t = t_ref[:]
    k = lax.broadcasted_iota(jnp.int32, (1, HALF), 1).astype(jnp.float32)
    inv_freq = jnp.exp(k * (-2.0 * jnp.log(100.0) / D))
    arg = t * inv_freq
    temb_ref[:, :HALF] = jnp.sin(arg)
    temb_ref[:, HALF:] = jnp.cos(arg)
    cond_ref[:] = lax.dot_general(
        c_ref[:], w_ref[:], (((1,), (1,)), ((), ())),
        preferred_element_type=jnp.float32)


def _tc_call(t2, cond_emb, w):
    return pl.pallas_call(
        _tc_body,
        grid=(B // TC_BLK,),
        in_specs=[
            pl.BlockSpec((TC_BLK, 1), lambda i: (i, 0)),
            pl.BlockSpec((TC_BLK, D), lambda i: (i, 0)),
            pl.BlockSpec((D, D), lambda i: (0, 0)),
        ],
        out_specs=[
            pl.BlockSpec((TC_BLK, D), lambda i: (i, 0)),
            pl.BlockSpec((TC_BLK, D), lambda i: (i, 0)),
        ],
        out_shape=[
            jax.ShapeDtypeStruct((B, D), jnp.float32),
            jax.ShapeDtypeStruct((B, D), jnp.float32),
        ],
    )(t2, cond_emb, w)


_mesh = plsc.VectorSubcoreMesh(core_axis_name="c", subcore_axis_name="s")


@functools.partial(
    pl.kernel,
    mesh=_mesh,
    out_type=jax.ShapeDtypeStruct((B, SG, D), jnp.float32),
    scratch_types=[
        pltpu.VMEM((ROWS_PER_W * SRUN,), jnp.int32),
        pltpu.VMEM((ROWS_PER_W * D,), jnp.float32),
        pltpu.VMEM((SG, D), jnp.float32),
        pltpu.VMEM((SG, D), jnp.float32),
        pltpu.VMEM((TAIL, D), jnp.float32),
        pltpu.VMEM((TAIL, D), jnp.float32),
        pltpu.SemaphoreType.DMA,
        pltpu.SemaphoreType.DMA,
        pltpu.SemaphoreType.DMA,
        pltpu.SemaphoreType.DMA,
        pltpu.SemaphoreType.DMA,
        pltpu.SemaphoreType.DMA,
    ],
)
def _sc_gather(xg_hbm, temb1_hbm, table_hbm, out_hbm,
               idxs_v, temb_v, bw0, bw1, bt0, bt1, g0, g1, t0, t1, w0, w1):
    wid = lax.axis_index("s") * 2 + lax.axis_index("c")
    base = wid * ROWS_PER_W
    bufw = (bw0, bw1)
    buft = (bt0, bt1)
    gsems = (g0, g1)
    tsems = (t0, t1)
    wsems = (w0, w1)

    pltpu.sync_copy(xg_hbm.at[pl.ds(base * SRUN, ROWS_PER_W * SRUN)], idxs_v)
    pltpu.sync_copy(temb1_hbm.at[pl.ds(base * D, ROWS_PER_W * D)], temb_v)

    def issue_gathers(i, p):
        pltpu.async_copy(table_hbm.at[idxs_v.at[pl.ds(i * SRUN, SG)]],
                         bufw[p], gsems[p])
        pltpu.async_copy(table_hbm.at[idxs_v.at[pl.ds(i * SRUN + 56, TAIL)]],
                         buft[p], tsems[p])

    def wait_write(p):
        pltpu.make_async_copy(bufw[p], out_hbm.at[0], wsems[p]).wait()

    def drain_and_write(i, p):
        pltpu.make_async_copy(table_hbm.at[idxs_v.at[pl.ds(0, SG)]],
                              bufw[p], gsems[p]).wait()
        pltpu.make_async_copy(table_hbm.at[idxs_v.at[pl.ds(0, TAIL)]],
                              buft[p], tsems[p]).wait()
        for r in range(3):
            for c in range(D // 16):
                bufw[p][48 + r, pl.ds(c * 16, 16)] = \
                    buft[p][r, pl.ds(c * 16, 16)]
        for c in range(D // 16):
            bufw[p][0, pl.ds(c * 16, 16)] = temb_v[pl.ds(i * D + c * 16, 16)]
        pltpu.async_copy(bufw[p], out_hbm.at[base + i], wsems[p])

    issue_gathers(0, 0)
    issue_gathers(1, 1)

    @pl.loop(0, ROWS_PER_W - 2, step=2)
    def _(g):
        drain_and_write(g, 0)
        drain_and_write(g + 1, 1)
        wait_write(0)
        issue_gathers(g + 2, 0)
        wait_write(1)
        issue_gathers(g + 3, 1)

    drain_and_write(ROWS_PER_W - 2, 0)
    drain_and_write(ROWS_PER_W - 1, 1)
    wait_write(0)
    wait_write(1)


def kernel(x, t, condition_emb, emb_table, cond_W):
    x2 = x.astype(jnp.int32)
    d5 = jnp.tile(x2[:, :1], (1, 5))
    xg = jnp.concatenate([x2[:, :1], x2, d5, x2[:, 47:50], d5], axis=1)
    xg1 = xg.reshape(-1)
    temb, cond = _tc_call(t.reshape(B, 1), condition_emb, cond_W)
    temb1 = temb.reshape(-1)
    dit = _sc_gather(xg1, temb1, emb_table)
    return dit, cond


# speedup vs baseline: 2.8763x; 1.9276x over previous
import functools

import jax
import jax.numpy as jnp
from jax import lax
from jax.experimental import pallas as pl
from jax.experimental.pallas import tpu as pltpu
from jax.experimental.pallas import tpu_sc as plsc

D = 768
HALF = D // 2
B = 1024
S = 50
SG = S + 1
BLK = 32
NW = 32
UNITS = SG
NBUF = 3
TC_BLK = 256


def _tc_body(t_ref, c_ref, w_ref, temb_ref, cond_ref):
    t = t_ref[:]
    k = lax.broadcasted_iota(jnp.int32, (1, HALF), 1).astype(jnp.float32)
    inv_freq = jnp.exp(k * (-2.0 * jnp.log(100.0) / D))
    arg = t * inv_freq
    temb_ref[:, :HALF] = jnp.sin(arg)
    temb_ref[:, HALF:] = jnp.cos(arg)
    cond_ref[:] = lax.dot_general(
        c_ref[:], w_ref[:], (((1,), (1,)), ((), ())),
        preferred_element_type=jnp.float32)


def _tc_call(t2, cond_emb, w):
    return pl.pallas_call(
        _tc_body,
        grid=(B // TC_BLK,),
        in_specs=[
            pl.BlockSpec((TC_BLK, 1), lambda i: (i, 0)),
            pl.BlockSpec((TC_BLK, D), lambda i: (i, 0)),
            pl.BlockSpec((D, D), lambda i: (0, 0)),
        ],
        out_specs=[
            pl.BlockSpec((TC_BLK, D), lambda i: (i, 0)),
            pl.BlockSpec((TC_BLK, D), lambda i: (i, 0)),
        ],
        out_shape=[
            jax.ShapeDtypeStruct((B, D), jnp.float32),
            jax.ShapeDtypeStruct((B, D), jnp.float32),
        ],
    )(t2, cond_emb, w)


_mesh = plsc.VectorSubcoreMesh(core_axis_name="c", subcore_axis_name="s")


@functools.partial(
    pl.kernel,
    mesh=_mesh,
    out_type=jax.ShapeDtypeStruct((SG, B, D), jnp.float32),
    scratch_types=[
        pltpu.VMEM((S * BLK,), jnp.int32),
        pltpu.VMEM((BLK, D), jnp.float32),
        pltpu.VMEM((BLK, D), jnp.float32),
        pltpu.VMEM((BLK, D), jnp.float32),
        pltpu.SemaphoreType.DMA,
        pltpu.SemaphoreType.DMA,
        pltpu.SemaphoreType.DMA,
        pltpu.SemaphoreType.DMA,
        pltpu.SemaphoreType.DMA,
        pltpu.SemaphoreType.DMA,
    ],
)
def _sc_gather(xt_hbm, temb_hbm, table_hbm, out_hbm,
               idxs_v, b0, b1, b2, g0, g1, g2, w0, w1, w2):
    wid = lax.axis_index("s") * 2 + lax.axis_index("c")
    bufs = (b0, b1, b2)
    gsems = (g0, g1, g2)
    wsems = (w0, w1, w2)

    pltpu.sync_copy(xt_hbm.at[pl.ds(wid * S * BLK, S * BLK)], idxs_v)

    def issue_load(m, p):
        @pl.when(m == 0)
        def _():
            pltpu.async_copy(temb_hbm.at[pl.ds(wid * BLK, BLK)],
                             bufs[p], gsems[p])

        @pl.when(m > 0)
        def _():
            pltpu.async_copy(
                table_hbm.at[idxs_v.at[pl.ds((m - 1) * BLK, BLK)]],
                bufs[p], gsems[p])

    def drain_and_write(m, p):
        pltpu.make_async_copy(temb_hbm.at[pl.ds(0, BLK)],
                              bufs[p], gsems[p]).wait()
        u = wid * S + (m - 1)
        plane = jnp.where(m == 0, 0, u // 32 + 1)
        row = jnp.where(m == 0, wid, u % 32)
        pltpu.async_copy(bufs[p], out_hbm.at[plane, pl.ds(row * BLK, BLK)],
                         wsems[p])

    def wait_write(p):
        pltpu.make_async_copy(bufs[p], out_hbm.at[0, pl.ds(0, BLK)],
                              wsems[p]).wait()

    for p in range(NBUF):
        issue_load(p, p)

    @pl.loop(0, UNITS - NBUF, step=NBUF)
    def _(g):
        for p in range(NBUF):
            drain_and_write(g + p, p)
        for p in range(NBUF):
            wait_write(p)
            issue_load(g + NBUF + p, p)

    for m in (UNITS - 3, UNITS - 2, UNITS - 1):
        drain_and_write(m, m % NBUF)
    for m in (UNITS - 3, UNITS - 2, UNITS - 1):
        wait_write(m % NBUF)


def kernel(x, t, condition_emb, emb_table, cond_W):
    xt1 = x.astype(jnp.int32).T.reshape(-1)
    temb, cond = _tc_call(t.reshape(B, 1), condition_emb, cond_W)
    dit = _sc_gather(xt1, temb, emb_table)
    return dit.transpose(1, 0, 2), cond
